# Initial kernel scaffold; baseline (speedup 1.0000x reference)
import jax
import jax.numpy as jnp
from jax.experimental import pallas as pl


def kernel(node_features, edge_index, edge_features, Wq, bq, Wk, bk, Wv, bv, Wb1, bb1, Wb2, bb2, Wo, bo, Wf1, bf1, Wf2, bf2, g1, be1, g2, be2):
    raise NotImplementedError("write your pallas kernel here")


# speedup vs baseline: 8.1527x; 8.1527x over previous
import functools

import jax
import jax.numpy as jnp
import numpy as np
from jax import lax
from jax.experimental import pallas as pl
from jax.experimental.pallas import tpu as pltpu
from jax.experimental.pallas import tpu_sc as plsc

N = 10000
E = 320000
D = 128
H = 4
DH = D // H
DE = 16
SCALE = 1.0 / (DH ** 0.5)

NP = 10240
NW = 32
CHUNK = 128
NCHUNK = 80
EPW = CHUNK * NCHUNK
EP = EPW * NW
RPS = NP // 16
W = D // 16
HW = W // 2

_COLS = [32 * h + 4 * j + t for j in range(W) for h in range(H) for t in range(4)]

_mesh = plsc.VectorSubcoreMesh(core_axis_name="c", subcore_axis_name="s")
_sc_params = pltpu.CompilerParams(use_tc_tiling_on_sc=False)


def _silu(x):
    return x * (1.0 / (1.0 + jnp.exp(-x)))


def _ln(x, g, b):
    mu = jnp.mean(x, axis=-1, keepdims=True)
    var = jnp.mean((x - mu) ** 2, axis=-1, keepdims=True)
    return (x - mu) / jnp.sqrt(var + 1e-5) * g + b


def _qkv_body(x_ref, wq_ref, bq_ref, wk_ref, bk_ref, wv_ref, bv_ref,
              q_ref, k_ref, v_ref):
    xb = x_ref[...]
    q_ref[...] = jnp.dot(xb, wq_ref[...], preferred_element_type=jnp.float32) + bq_ref[...][None, :]
    k_ref[...] = jnp.dot(xb, wk_ref[...], preferred_element_type=jnp.float32) + bk_ref[...][None, :]
    v_ref[...] = jnp.dot(xb, wv_ref[...], preferred_element_type=jnp.float32) + bv_ref[...][None, :]


def _tc_qkv(x_pad, Wq, bq, Wk, bk, Wv, bv):
    blk = 512
    grid = NP // blk
    whole2 = pl.BlockSpec((D, D), lambda i: (0, 0))
    whole1 = pl.BlockSpec((D,), lambda i: (0,))
    rows = pl.BlockSpec((blk, D), lambda i: (i, 0))
    return pl.pallas_call(
        _qkv_body,
        grid=(grid,),
        in_specs=[rows, whole2, whole1, whole2, whole1, whole2, whole1],
        out_specs=[rows, rows, rows],
        out_shape=[jax.ShapeDtypeStruct((NP, D), jnp.float32)] * 3,
    )(x_pad, Wq, bq, Wk, bk, Wv, bv)


def _bias_body(ef_ref, w1_ref, b1_ref, w2_ref, b2_ref, o_ref):
    h1 = jnp.dot(ef_ref[...], w1_ref[...], preferred_element_type=jnp.float32) + b1_ref[...][None, :]
    h1 = _silu(h1)
    b = jnp.dot(h1, w2_ref[...], preferred_element_type=jnp.float32) + b2_ref[...][None, :]
    o_ref[...] = jnp.concatenate(
        [b, jnp.zeros((b.shape[0], 16 - H), jnp.float32)], axis=1)


def _tc_bias(ef_pad, Wb1, bb1, Wb2, bb2):
    blk = 4096
    grid = EP // blk
    return pl.pallas_call(
        _bias_body,
        grid=(grid,),
        in_specs=[
            pl.BlockSpec((blk, DE), lambda i: (i, 0)),
            pl.BlockSpec((DE, H), lambda i: (0, 0)),
            pl.BlockSpec((H,), lambda i: (0,)),
            pl.BlockSpec((H, H), lambda i: (0, 0)),
            pl.BlockSpec((H,), lambda i: (0,)),
        ],
        out_specs=pl.BlockSpec((blk, 16), lambda i: (i, 0)),
        out_shape=jax.ShapeDtypeStruct((EP, 16), jnp.float32),
    )(ef_pad, Wb1, bb1, Wb2, bb2)


def _final_body(a0_ref, a1_ref, x_ref, wo_ref, bo_ref, wf1_ref, bf1_ref,
                wf2_ref, bf2_ref, g1_ref, be1_ref, g2_ref, be2_ref, o_ref):
    agg = a0_ref[...] + a1_ref[...]
    o = jnp.dot(agg, wo_ref[...], preferred_element_type=jnp.float32) + bo_ref[...][None, :]
    x1 = _ln(x_ref[...] + o, g1_ref[...][None, :], be1_ref[...][None, :])
    h = _silu(jnp.dot(x1, wf1_ref[...], preferred_element_type=jnp.float32) + bf1_ref[...][None, :])
    f = jnp.dot(h, wf2_ref[...], preferred_element_type=jnp.float32) + bf2_ref[...][None, :]
    o_ref[...] = _ln(x1 + f, g2_ref[...][None, :], be2_ref[...][None, :])


def _tc_final(a0, a1, x_pad, Wo, bo, Wf1, bf1, Wf2, bf2, g1, be1, g2, be2):
    blk = 512
    grid = NP // blk
    rows = pl.BlockSpec((blk, D), lambda i: (i, 0))
    w128 = pl.BlockSpec((D,), lambda i: (0,))
    return pl.pallas_call(
        _final_body,
        grid=(grid,),
        in_specs=[
            rows, rows, rows,
            pl.BlockSpec((D, D), lambda i: (0, 0)), w128,
            pl.BlockSpec((D, 2 * D), lambda i: (0, 0)),
            pl.BlockSpec((2 * D,), lambda i: (0,)),
            pl.BlockSpec((2 * D, D), lambda i: (0, 0)), w128,
            w128, w128, w128, w128,
        ],
        out_specs=rows,
        out_shape=jax.ShapeDtypeStruct((NP, D), jnp.float32),
    )(a0, a1, x_pad, Wo, bo, Wf1, bf1, Wf2, bf2, g1, be1, g2, be2)


def _take(x, idx):
    return jnp.take_along_axis(x, idx, axis=0, mode="promise_in_bounds")


def _lane_perms():
    lane = lax.iota(jnp.int32, 16)
    swap1 = lane ^ 1
    swap2 = lane ^ 2
    heads = (lane & 3) * 4
    bcast = lane >> 2
    return lane, swap1, swap2, heads, bcast


def _pass_a_body(dst_hbm, src_hbm, q_hbm, k_hbm, bias_hbm,
                 ex_hbm, nrm0_hbm, nrm1_hbm,
                 dstv, srcv, qrows, krows, brows, exv, stage, nrm_sh, sem):
    c = lax.axis_index("c")
    s = lax.axis_index("s")
    wid = s * 2 + c

    def zero_row(r, carry):
        stage[r, :] = jnp.zeros((16,), jnp.float32)
        return carry
    lax.fori_loop(0, RPS, zero_row, 0)
    pltpu.sync_copy(stage, nrm_sh.at[pl.ds(s * RPS, RPS), :])
    plsc.subcore_barrier()

    lane, swap1, swap2, heads, _ = _lane_perms()

    def chunk_body(ci, carry):
        base = wid * EPW + ci * CHUNK
        pltpu.sync_copy(dst_hbm.at[pl.ds(base, CHUNK)], dstv)
        pltpu.sync_copy(src_hbm.at[pl.ds(base, CHUNK)], srcv)
        pltpu.async_copy(q_hbm.at[dstv], qrows, sem).wait()
        pltpu.async_copy(k_hbm.at[srcv], krows, sem).wait()
        pltpu.sync_copy(bias_hbm.at[pl.ds(base, CHUNK), :], brows)

        def edge_body(e, ecarry):
            acc = qrows[e, pl.ds(0, 16)] * krows[e, pl.ds(0, 16)]
            for j in range(1, W):
                acc = acc + (qrows[e, pl.ds(16 * j, 16)]
                             * krows[e, pl.ds(16 * j, 16)])
            acc = acc + _take(acc, swap1)
            acc = acc + _take(acc, swap2)
            logits = _take(acc, heads) * SCALE + brows[e, :]
            exv[e, :] = jnp.where(lane < H, jnp.exp(logits), 0.0)
            return ecarry

        lax.fori_loop(0, CHUNK, edge_body, 0)
        pltpu.sync_copy(exv, ex_hbm.at[pl.ds(base, CHUNK), :])
        pltpu.sync_copy(exv, nrm_sh.at[dstv], add=True)
        return carry

    lax.fori_loop(0, NCHUNK, chunk_body, 0)
    plsc.subcore_barrier()

    pltpu.sync_copy(nrm_sh.at[pl.ds(s * RPS, RPS), :], stage)

    @pl.when(c == 0)
    def _():
        pltpu.sync_copy(stage, nrm0_hbm.at[pl.ds(s * RPS, RPS), :])

    @pl.when(c == 1)
    def _():
        pltpu.sync_copy(stage, nrm1_hbm.at[pl.ds(s * RPS, RPS), :])


_sc_pass_a = functools.partial(
    pl.kernel,
    _pass_a_body,
    out_type=(jax.ShapeDtypeStruct((EP, 16), jnp.float32),
              jax.ShapeDtypeStruct((NP, 16), jnp.float32),
              jax.ShapeDtypeStruct((NP, 16), jnp.float32)),
    mesh=_mesh,
    scratch_types=[
        pltpu.VMEM((CHUNK,), jnp.int32),
        pltpu.VMEM((CHUNK,), jnp.int32),
        pltpu.VMEM((CHUNK, D), jnp.float32),
        pltpu.VMEM((CHUNK, D), jnp.float32),
        pltpu.VMEM((CHUNK, 16), jnp.float32),
        pltpu.VMEM((CHUNK, 16), jnp.float32),
        pltpu.VMEM((RPS, 16), jnp.float32),
        pltpu.VMEM_SHARED((NP, 16), jnp.float32),
        pltpu.SemaphoreType.DMA,
    ],
    compiler_params=_sc_params,
)()


def _pass_b_body(dst_hbm, src_hbm, vlo_hbm, vhi_hbm, ex_hbm, nrm0_hbm,
                 nrm1_hbm,
                 a0lo_hbm, a0hi_hbm, a1lo_hbm, a1hi_hbm,
                 dstv, srcv, vrows, wrows, exr, n0r, n1r, stage, agg_sh, sem):
    c = lax.axis_index("c")
    s = lax.axis_index("s")
    wid = s * 2 + c

    _, _, _, _, bcast = _lane_perms()

    for v_hbm, out0_hbm, out1_hbm in ((vlo_hbm, a0lo_hbm, a1lo_hbm),
                                      (vhi_hbm, a0hi_hbm, a1hi_hbm)):
        def zero_row(r, carry):
            for j in range(HW):
                stage[r, j, :] = jnp.zeros((16,), jnp.float32)
            return carry
        lax.fori_loop(0, RPS, zero_row, 0)
        pltpu.sync_copy(stage, agg_sh.at[pl.ds(s * RPS, RPS), :, :])
        plsc.subcore_barrier()

        def chunk_body(ci, carry):
            base = wid * EPW + ci * CHUNK
            pltpu.sync_copy(dst_hbm.at[pl.ds(base, CHUNK)], dstv)
            pltpu.sync_copy(src_hbm.at[pl.ds(base, CHUNK)], srcv)
            pltpu.async_copy(v_hbm.at[srcv], vrows, sem).wait()
            pltpu.sync_copy(ex_hbm.at[pl.ds(base, CHUNK), :], exr)
            pltpu.async_copy(nrm0_hbm.at[dstv], n0r, sem).wait()
            pltpu.async_copy(nrm1_hbm.at[dstv], n1r, sem).wait()

            def edge_body(e, ecarry):
                nrm = n0r[e, :] + n1r[e, :]
                attn = exr[e, :] / jnp.maximum(nrm, 1e-12)
                att_b = _take(attn, bcast)
                for j in range(HW):
                    wrows[e, j, :] = vrows[e, j, :] * att_b
                return ecarry

            lax.fori_loop(0, CHUNK, edge_body, 0)
            pltpu.sync_copy(wrows, agg_sh.at[dstv], add=True)
            return carry

        lax.fori_loop(0, NCHUNK, chunk_body, 0)
        plsc.subcore_barrier()

        pltpu.sync_copy(agg_sh.at[pl.ds(s * RPS, RPS), :, :], stage)

        @pl.when(c == 0)
        def _():
            pltpu.sync_copy(stage, out0_hbm.at[pl.ds(s * RPS, RPS), :, :])

        @pl.when(c == 1)
        def _():
            pltpu.sync_copy(stage, out1_hbm.at[pl.ds(s * RPS, RPS), :, :])


_sc_pass_b = functools.partial(
    pl.kernel,
    _pass_b_body,
    out_type=tuple(jax.ShapeDtypeStruct((NP, HW, 16), jnp.float32)
                   for _ in range(4)),
    mesh=_mesh,
    scratch_types=[
        pltpu.VMEM((CHUNK,), jnp.int32),
        pltpu.VMEM((CHUNK,), jnp.int32),
        pltpu.VMEM((CHUNK, HW, 16), jnp.float32),
        pltpu.VMEM((CHUNK, HW, 16), jnp.float32),
        pltpu.VMEM((CHUNK, 16), jnp.float32),
        pltpu.VMEM((CHUNK, 16), jnp.float32),
        pltpu.VMEM((CHUNK, 16), jnp.float32),
        pltpu.VMEM((RPS, HW, 16), jnp.float32),
        pltpu.VMEM_SHARED((NP, HW, 16), jnp.float32),
        pltpu.SemaphoreType.DMA,
    ],
    compiler_params=_sc_params,
)()


def kernel(node_features, edge_index, edge_features, Wq, bq, Wk, bk, Wv, bv,
           Wb1, bb1, Wb2, bb2, Wo, bo, Wf1, bf1, Wf2, bf2, g1, be1, g2, be2):
    cols = np.array(_COLS, np.int32)
    x_pad = jnp.zeros((NP, D), jnp.float32).at[:N].set(node_features)
    pad_e = EP - E
    src_pad = jnp.concatenate([edge_index[0], jnp.zeros((pad_e,), jnp.int32)])
    dst_pad = jnp.concatenate([edge_index[1], jnp.full((pad_e,), N, jnp.int32)])
    ef_pad = jnp.zeros((EP, DE), jnp.float32).at[:E].set(edge_features)

    q, k, v = _tc_qkv(x_pad, Wq[:, cols], bq[cols], Wk[:, cols], bk[cols],
                      Wv[:, cols], bv[cols])
    v = v.reshape(NP, W, 16)
    bias = _tc_bias(ef_pad, Wb1, bb1, Wb2, bb2)

    ex, nrm0, nrm1 = _sc_pass_a(dst_pad, src_pad, q, k, bias)
    a0lo, a0hi, a1lo, a1hi = _sc_pass_b(
        dst_pad, src_pad, v[:, :HW, :], v[:, HW:, :], ex, nrm0, nrm1)
    agg0 = jnp.concatenate([a0lo.reshape(NP, D // 2), a0hi.reshape(NP, D // 2)], axis=1)
    agg1 = jnp.concatenate([a1lo.reshape(NP, D // 2), a1hi.reshape(NP, D // 2)], axis=1)

    out = _tc_final(agg0, agg1, x_pad,
                    Wo[cols, :], bo, Wf1, bf1, Wf2, bf2, g1, be1, g2, be2)
    return out[:N]


# speedup vs baseline: 10.4603x; 1.2830x over previous
import functools

import jax
import jax.numpy as jnp
import numpy as np
from jax import lax
from jax.experimental import pallas as pl
from jax.experimental.pallas import tpu as pltpu
from jax.experimental.pallas import tpu_sc as plsc

N = 10000
E = 320000
D = 128
H = 4
DH = D // H
DE = 16
SCALE = 1.0 / (DH ** 0.5)

NP = 10240
NW = 32
CHUNK = 128
NCHUNK = 80
EPW = CHUNK * NCHUNK
EP = EPW * NW
RPS = NP // 16
W = D // 16
HW = W // 2

_COLS = [32 * h + 4 * j + t for j in range(W) for h in range(H) for t in range(4)]

_mesh = plsc.VectorSubcoreMesh(core_axis_name="c", subcore_axis_name="s")
_sc_params = pltpu.CompilerParams(use_tc_tiling_on_sc=False)


def _silu(x):
    return x * (1.0 / (1.0 + jnp.exp(-x)))


def _ln(x, g, b):
    mu = jnp.mean(x, axis=-1, keepdims=True)
    var = jnp.mean((x - mu) ** 2, axis=-1, keepdims=True)
    return (x - mu) / jnp.sqrt(var + 1e-5) * g + b


def _qkv_body(x_ref, wq_ref, bq_ref, wk_ref, bk_ref, wv_ref, bv_ref,
              q_ref, k_ref, v_ref):
    xb = x_ref[...]
    q_ref[...] = jnp.dot(xb, wq_ref[...], preferred_element_type=jnp.float32) + bq_ref[...][None, :]
    k_ref[...] = jnp.dot(xb, wk_ref[...], preferred_element_type=jnp.float32) + bk_ref[...][None, :]
    v_ref[...] = jnp.dot(xb, wv_ref[...], preferred_element_type=jnp.float32) + bv_ref[...][None, :]


def _tc_qkv(x_pad, Wq, bq, Wk, bk, Wv, bv):
    blk = 512
    grid = NP // blk
    whole2 = pl.BlockSpec((D, D), lambda i: (0, 0))
    whole1 = pl.BlockSpec((D,), lambda i: (0,))
    rows = pl.BlockSpec((blk, D), lambda i: (i, 0))
    return pl.pallas_call(
        _qkv_body,
        grid=(grid,),
        in_specs=[rows, whole2, whole1, whole2, whole1, whole2, whole1],
        out_specs=[rows, rows, rows],
        out_shape=[jax.ShapeDtypeStruct((NP, D), jnp.float32)] * 3,
    )(x_pad, Wq, bq, Wk, bk, Wv, bv)


def _bias_body(ef_ref, w1_ref, b1_ref, w2_ref, b2_ref, o_ref):
    h1 = jnp.dot(ef_ref[...], w1_ref[...], preferred_element_type=jnp.float32) + b1_ref[...][None, :]
    h1 = _silu(h1)
    b = jnp.dot(h1, w2_ref[...], preferred_element_type=jnp.float32) + b2_ref[...][None, :]
    o_ref[...] = jnp.concatenate(
        [b, jnp.zeros((b.shape[0], 16 - H), jnp.float32)], axis=1)


def _tc_bias(ef_pad, Wb1, bb1, Wb2, bb2):
    blk = 4096
    grid = EP // blk
    return pl.pallas_call(
        _bias_body,
        grid=(grid,),
        in_specs=[
            pl.BlockSpec((blk, DE), lambda i: (i, 0)),
            pl.BlockSpec((DE, H), lambda i: (0, 0)),
            pl.BlockSpec((H,), lambda i: (0,)),
            pl.BlockSpec((H, H), lambda i: (0, 0)),
            pl.BlockSpec((H,), lambda i: (0,)),
        ],
        out_specs=pl.BlockSpec((blk, 16), lambda i: (i, 0)),
        out_shape=jax.ShapeDtypeStruct((EP, 16), jnp.float32),
    )(ef_pad, Wb1, bb1, Wb2, bb2)


def _final_body(a0_ref, a1_ref, x_ref, wo_ref, bo_ref, wf1_ref, bf1_ref,
                wf2_ref, bf2_ref, g1_ref, be1_ref, g2_ref, be2_ref, o_ref):
    agg = a0_ref[...] + a1_ref[...]
    o = jnp.dot(agg, wo_ref[...], preferred_element_type=jnp.float32) + bo_ref[...][None, :]
    x1 = _ln(x_ref[...] + o, g1_ref[...][None, :], be1_ref[...][None, :])
    h = _silu(jnp.dot(x1, wf1_ref[...], preferred_element_type=jnp.float32) + bf1_ref[...][None, :])
    f = jnp.dot(h, wf2_ref[...], preferred_element_type=jnp.float32) + bf2_ref[...][None, :]
    o_ref[...] = _ln(x1 + f, g2_ref[...][None, :], be2_ref[...][None, :])


def _tc_final(a0, a1, x_pad, Wo, bo, Wf1, bf1, Wf2, bf2, g1, be1, g2, be2):
    blk = 512
    grid = NP // blk
    rows = pl.BlockSpec((blk, D), lambda i: (i, 0))
    w128 = pl.BlockSpec((D,), lambda i: (0,))
    return pl.pallas_call(
        _final_body,
        grid=(grid,),
        in_specs=[
            rows, rows, rows,
            pl.BlockSpec((D, D), lambda i: (0, 0)), w128,
            pl.BlockSpec((D, 2 * D), lambda i: (0, 0)),
            pl.BlockSpec((2 * D,), lambda i: (0,)),
            pl.BlockSpec((2 * D, D), lambda i: (0, 0)), w128,
            w128, w128, w128, w128,
        ],
        out_specs=rows,
        out_shape=jax.ShapeDtypeStruct((NP, D), jnp.float32),
    )(a0, a1, x_pad, Wo, bo, Wf1, bf1, Wf2, bf2, g1, be1, g2, be2)


def _take(x, idx):
    return jnp.take_along_axis(x, idx, axis=0, mode="promise_in_bounds")


def _lane_perms():
    lane = lax.iota(jnp.int32, 16)
    swap1 = lane ^ 1
    swap2 = lane ^ 2
    heads = (lane & 3) * 4
    bcast = lane >> 2
    return lane, swap1, swap2, heads, bcast


def _pass_a_body(dst_hbm, src_hbm, q_hbm, k_hbm, bias_hbm,
                 ex_hbm, nrm0_hbm, nrm1_hbm,
                 dstv, srcv, qrows, krows, brows, exv, stage, nrm_sh, sem):
    c = lax.axis_index("c")
    s = lax.axis_index("s")
    wid = s * 2 + c

    def zero_row(r, carry):
        stage[r, :] = jnp.zeros((16,), jnp.float32)
        return carry
    lax.fori_loop(0, RPS, zero_row, 0)
    pltpu.sync_copy(stage, nrm_sh.at[pl.ds(s * RPS, RPS), :])
    plsc.subcore_barrier()

    lane, swap1, swap2, heads, _ = _lane_perms()

    def chunk_body(ci, carry):
        base = wid * EPW + ci * CHUNK
        pltpu.sync_copy(dst_hbm.at[pl.ds(base, CHUNK)], dstv)
        pltpu.sync_copy(src_hbm.at[pl.ds(base, CHUNK)], srcv)
        dq = pltpu.async_copy(q_hbm.at[dstv], qrows, sem)
        dk = pltpu.async_copy(k_hbm.at[srcv], krows, sem)
        db = pltpu.async_copy(bias_hbm.at[pl.ds(base, CHUNK), :], brows, sem)
        dq.wait()
        dk.wait()
        db.wait()

        def edge_body(i, ecarry):
            for u in range(2):
                e = i * 2 + u
                acc = qrows[e, pl.ds(0, 16)] * krows[e, pl.ds(0, 16)]
                for j in range(1, W):
                    acc = acc + (qrows[e, pl.ds(16 * j, 16)]
                                 * krows[e, pl.ds(16 * j, 16)])
                acc = acc + _take(acc, swap1)
                acc = acc + _take(acc, swap2)
                logits = _take(acc, heads) * SCALE + brows[e, :]
                exv[e, :] = jnp.where(lane < H, jnp.exp(logits), 0.0)
            return ecarry

        lax.fori_loop(0, CHUNK // 2, edge_body, 0)
        pltpu.sync_copy(exv, ex_hbm.at[pl.ds(base, CHUNK), :])
        pltpu.sync_copy(exv, nrm_sh.at[dstv], add=True)
        return carry

    lax.fori_loop(0, NCHUNK, chunk_body, 0)
    plsc.subcore_barrier()

    pltpu.sync_copy(nrm_sh.at[pl.ds(s * RPS, RPS), :], stage)

    @pl.when(c == 0)
    def _():
        pltpu.sync_copy(stage, nrm0_hbm.at[pl.ds(s * RPS, RPS), :])

    @pl.when(c == 1)
    def _():
        pltpu.sync_copy(stage, nrm1_hbm.at[pl.ds(s * RPS, RPS), :])


_sc_pass_a = functools.partial(
    pl.kernel,
    _pass_a_body,
    out_type=(jax.ShapeDtypeStruct((EP, 16), jnp.float32),
              jax.ShapeDtypeStruct((NP, 16), jnp.float32),
              jax.ShapeDtypeStruct((NP, 16), jnp.float32)),
    mesh=_mesh,
    scratch_types=[
        pltpu.VMEM((CHUNK,), jnp.int32),
        pltpu.VMEM((CHUNK,), jnp.int32),
        pltpu.VMEM((CHUNK, D), jnp.float32),
        pltpu.VMEM((CHUNK, D), jnp.float32),
        pltpu.VMEM((CHUNK, 16), jnp.float32),
        pltpu.VMEM((CHUNK, 16), jnp.float32),
        pltpu.VMEM((RPS, 16), jnp.float32),
        pltpu.VMEM_SHARED((NP, 16), jnp.float32),
        pltpu.SemaphoreType.DMA,
    ],
    compiler_params=_sc_params,
)()


def _pass_b_body(dst_hbm, src_hbm, vlo_hbm, vhi_hbm, ex_hbm, nrm0_hbm,
                 nrm1_hbm,
                 a0lo_hbm, a0hi_hbm, a1lo_hbm, a1hi_hbm,
                 dstv, srcv, vrows, wrows, exr, n0r, n1r, stage, agg_sh, sem):
    c = lax.axis_index("c")
    s = lax.axis_index("s")
    wid = s * 2 + c

    _, _, _, _, bcast = _lane_perms()

    for v_hbm, out0_hbm, out1_hbm in ((vlo_hbm, a0lo_hbm, a1lo_hbm),
                                      (vhi_hbm, a0hi_hbm, a1hi_hbm)):
        def zero_row(r, carry):
            for j in range(HW):
                stage[r, j, :] = jnp.zeros((16,), jnp.float32)
            return carry
        lax.fori_loop(0, RPS, zero_row, 0)
        pltpu.sync_copy(stage, agg_sh.at[pl.ds(s * RPS, RPS), :, :])
        plsc.subcore_barrier()

        def chunk_body(ci, carry):
            base = wid * EPW + ci * CHUNK
            pltpu.sync_copy(dst_hbm.at[pl.ds(base, CHUNK)], dstv)
            pltpu.sync_copy(src_hbm.at[pl.ds(base, CHUNK)], srcv)
            dv = pltpu.async_copy(v_hbm.at[srcv], vrows, sem)
            de = pltpu.async_copy(ex_hbm.at[pl.ds(base, CHUNK), :], exr, sem)
            d0 = pltpu.async_copy(nrm0_hbm.at[dstv], n0r, sem)
            d1 = pltpu.async_copy(nrm1_hbm.at[dstv], n1r, sem)
            dv.wait()
            de.wait()
            d0.wait()
            d1.wait()

            def edge_body(i, ecarry):
                for u in range(2):
                    e = i * 2 + u
                    nrm = n0r[e, :] + n1r[e, :]
                    attn = exr[e, :] / jnp.maximum(nrm, 1e-12)
                    att_b = _take(attn, bcast)
                    for j in range(HW):
                        wrows[e, j, :] = vrows[e, j, :] * att_b
                return ecarry

            lax.fori_loop(0, CHUNK // 2, edge_body, 0)
            pltpu.sync_copy(wrows, agg_sh.at[dstv], add=True)
            return carry

        lax.fori_loop(0, NCHUNK, chunk_body, 0)
        plsc.subcore_barrier()

        pltpu.sync_copy(agg_sh.at[pl.ds(s * RPS, RPS), :, :], stage)

        @pl.when(c == 0)
        def _():
            pltpu.sync_copy(stage, out0_hbm.at[pl.ds(s * RPS, RPS), :, :])

        @pl.when(c == 1)
        def _():
            pltpu.sync_copy(stage, out1_hbm.at[pl.ds(s * RPS, RPS), :, :])


_sc_pass_b = functools.partial(
    pl.kernel,
    _pass_b_body,
    out_type=tuple(jax.ShapeDtypeStruct((NP, HW, 16), jnp.float32)
                   for _ in range(4)),
    mesh=_mesh,
    scratch_types=[
        pltpu.VMEM((CHUNK,), jnp.int32),
        pltpu.VMEM((CHUNK,), jnp.int32),
        pltpu.VMEM((CHUNK, HW, 16), jnp.float32),
        pltpu.VMEM((CHUNK, HW, 16), jnp.float32),
        pltpu.VMEM((CHUNK, 16), jnp.float32),
        pltpu.VMEM((CHUNK, 16), jnp.float32),
        pltpu.VMEM((CHUNK, 16), jnp.float32),
        pltpu.VMEM((RPS, HW, 16), jnp.float32),
        pltpu.VMEM_SHARED((NP, HW, 16), jnp.float32),
        pltpu.SemaphoreType.DMA,
    ],
    compiler_params=_sc_params,
)()


def kernel(node_features, edge_index, edge_features, Wq, bq, Wk, bk, Wv, bv,
           Wb1, bb1, Wb2, bb2, Wo, bo, Wf1, bf1, Wf2, bf2, g1, be1, g2, be2):
    cols = np.array(_COLS, np.int32)
    x_pad = jnp.zeros((NP, D), jnp.float32).at[:N].set(node_features)
    pad_e = EP - E
    src_pad = jnp.concatenate([edge_index[0], jnp.zeros((pad_e,), jnp.int32)])
    dst_pad = jnp.concatenate([edge_index[1], jnp.full((pad_e,), N, jnp.int32)])
    ef_pad = jnp.zeros((EP, DE), jnp.float32).at[:E].set(edge_features)

    q, k, v = _tc_qkv(x_pad, Wq[:, cols], bq[cols], Wk[:, cols], bk[cols],
                      Wv[:, cols], bv[cols])
    v = v.reshape(NP, W, 16)
    bias = _tc_bias(ef_pad, Wb1, bb1, Wb2, bb2)

    ex, nrm0, nrm1 = _sc_pass_a(dst_pad, src_pad, q, k, bias)
    a0lo, a0hi, a1lo, a1hi = _sc_pass_b(
        dst_pad, src_pad, v[:, :HW, :], v[:, HW:, :], ex, nrm0, nrm1)
    agg0 = jnp.concatenate([a0lo.reshape(NP, D // 2), a0hi.reshape(NP, D // 2)], axis=1)
    agg1 = jnp.concatenate([a1lo.reshape(NP, D // 2), a1hi.reshape(NP, D // 2)], axis=1)

    out = _tc_final(agg0, agg1, x_pad,
                    Wo[cols, :], bo, Wf1, bf1, Wf2, bf2, g1, be1, g2, be2)
    return out[:N]


# speedup vs baseline: 14.8657x; 1.4212x over previous
import functools

import jax
import jax.numpy as jnp
import numpy as np
from jax import lax
from jax.experimental import pallas as pl
from jax.experimental.pallas import tpu as pltpu
from jax.experimental.pallas import tpu_sc as plsc

N = 10000
E = 320000
D = 128
H = 4
DH = D // H
DE = 16
SCALE = 1.0 / (DH ** 0.5)

NP = 10240
NW = 32
CHUNK = 128
NCHUNK = 80
EPW = CHUNK * NCHUNK
EP = EPW * NW
RPS = NP // 16
W = D // 16
HW = W // 2

_COLS = [32 * h + 4 * j + t for j in range(W) for h in range(H) for t in range(4)]

_mesh = plsc.VectorSubcoreMesh(core_axis_name="c", subcore_axis_name="s")
_sc_params = pltpu.CompilerParams(use_tc_tiling_on_sc=False)


def _silu(x):
    return x * (1.0 / (1.0 + jnp.exp(-x)))


def _ln(x, g, b):
    mu = jnp.mean(x, axis=-1, keepdims=True)
    var = jnp.mean((x - mu) ** 2, axis=-1, keepdims=True)
    return (x - mu) / jnp.sqrt(var + 1e-5) * g + b


def _qkv_body(x_ref, wq_ref, bq_ref, wk_ref, bk_ref, wv_ref, bv_ref,
              q_ref, k_ref, v_ref):
    xb = x_ref[...]
    q_ref[...] = jnp.dot(xb, wq_ref[...], preferred_element_type=jnp.float32) + bq_ref[...][None, :]
    k_ref[...] = jnp.dot(xb, wk_ref[...], preferred_element_type=jnp.float32) + bk_ref[...][None, :]
    v_ref[...] = jnp.dot(xb, wv_ref[...], preferred_element_type=jnp.float32) + bv_ref[...][None, :]


def _tc_qkv(x_pad, Wq, bq, Wk, bk, Wv, bv):
    blk = 512
    grid = NP // blk
    whole2 = pl.BlockSpec((D, D), lambda i: (0, 0))
    whole1 = pl.BlockSpec((D,), lambda i: (0,))
    rows = pl.BlockSpec((blk, D), lambda i: (i, 0))
    return pl.pallas_call(
        _qkv_body,
        grid=(grid,),
        in_specs=[rows, whole2, whole1, whole2, whole1, whole2, whole1],
        out_specs=[rows, rows, rows],
        out_shape=[jax.ShapeDtypeStruct((NP, D), jnp.float32)] * 3,
    )(x_pad, Wq, bq, Wk, bk, Wv, bv)


def _bias_body(ef_ref, w1_ref, b1_ref, w2_ref, b2_ref, o_ref):
    h1 = jnp.dot(ef_ref[...], w1_ref[...], preferred_element_type=jnp.float32) + b1_ref[...][None, :]
    h1 = _silu(h1)
    b = jnp.dot(h1, w2_ref[...], preferred_element_type=jnp.float32) + b2_ref[...][None, :]
    o_ref[...] = jnp.concatenate(
        [b, jnp.zeros((b.shape[0], 16 - H), jnp.float32)], axis=1)


def _tc_bias(ef_pad, Wb1, bb1, Wb2, bb2):
    blk = 4096
    grid = EP // blk
    return pl.pallas_call(
        _bias_body,
        grid=(grid,),
        in_specs=[
            pl.BlockSpec((blk, DE), lambda i: (i, 0)),
            pl.BlockSpec((DE, H), lambda i: (0, 0)),
            pl.BlockSpec((H,), lambda i: (0,)),
            pl.BlockSpec((H, H), lambda i: (0, 0)),
            pl.BlockSpec((H,), lambda i: (0,)),
        ],
        out_specs=pl.BlockSpec((blk, 16), lambda i: (i, 0)),
        out_shape=jax.ShapeDtypeStruct((EP, 16), jnp.float32),
    )(ef_pad, Wb1, bb1, Wb2, bb2)


def _final_body(a0_ref, a1_ref, x_ref, wo_ref, bo_ref, wf1_ref, bf1_ref,
                wf2_ref, bf2_ref, g1_ref, be1_ref, g2_ref, be2_ref, o_ref):
    agg = a0_ref[...] + a1_ref[...]
    o = jnp.dot(agg, wo_ref[...], preferred_element_type=jnp.float32) + bo_ref[...][None, :]
    x1 = _ln(x_ref[...] + o, g1_ref[...][None, :], be1_ref[...][None, :])
    h = _silu(jnp.dot(x1, wf1_ref[...], preferred_element_type=jnp.float32) + bf1_ref[...][None, :])
    f = jnp.dot(h, wf2_ref[...], preferred_element_type=jnp.float32) + bf2_ref[...][None, :]
    o_ref[...] = _ln(x1 + f, g2_ref[...][None, :], be2_ref[...][None, :])


def _tc_final(a0, a1, x_pad, Wo, bo, Wf1, bf1, Wf2, bf2, g1, be1, g2, be2):
    blk = 512
    grid = NP // blk
    rows = pl.BlockSpec((blk, D), lambda i: (i, 0))
    w128 = pl.BlockSpec((D,), lambda i: (0,))
    return pl.pallas_call(
        _final_body,
        grid=(grid,),
        in_specs=[
            rows, rows, rows,
            pl.BlockSpec((D, D), lambda i: (0, 0)), w128,
            pl.BlockSpec((D, 2 * D), lambda i: (0, 0)),
            pl.BlockSpec((2 * D,), lambda i: (0,)),
            pl.BlockSpec((2 * D, D), lambda i: (0, 0)), w128,
            w128, w128, w128, w128,
        ],
        out_specs=rows,
        out_shape=jax.ShapeDtypeStruct((NP, D), jnp.float32),
    )(a0, a1, x_pad, Wo, bo, Wf1, bf1, Wf2, bf2, g1, be1, g2, be2)


def _take(x, idx):
    return jnp.take_along_axis(x, idx, axis=0, mode="promise_in_bounds")


def _lane_perms():
    lane = lax.iota(jnp.int32, 16)
    swap1 = lane ^ 1
    swap2 = lane ^ 2
    heads = (lane & 3) * 4
    bcast = lane >> 2
    return lane, swap1, swap2, heads, bcast


def _pass_a_body(dst_hbm, src_hbm, q_hbm, k_hbm, bias_hbm,
                 ex_hbm, nrm0_hbm, nrm1_hbm,
                 dstv0, srcv0, qrows0, krows0, brows0,
                 dstv1, srcv1, qrows1, krows1, brows1,
                 exv, stage, nrm_sh, sem0, sem1):
    c = lax.axis_index("c")
    s = lax.axis_index("s")
    wid = s * 2 + c

    def zero_row(r, carry):
        stage[r, :] = jnp.zeros((16,), jnp.float32)
        return carry
    lax.fori_loop(0, RPS, zero_row, 0)
    pltpu.sync_copy(stage, nrm_sh.at[pl.ds(s * RPS, RPS), :])
    plsc.subcore_barrier()

    lane, swap1, swap2, heads, _ = _lane_perms()
    bufs = ((dstv0, srcv0, qrows0, krows0, brows0, sem0),
            (dstv1, srcv1, qrows1, krows1, brows1, sem1))

    def fire(ci, b):
        dv, sv, qb, kb, bb, sm = b
        base = wid * EPW + ci * CHUNK
        pltpu.sync_copy(dst_hbm.at[pl.ds(base, CHUNK)], dv)
        pltpu.sync_copy(src_hbm.at[pl.ds(base, CHUNK)], sv)
        pltpu.async_copy(q_hbm.at[dv], qb, sm)
        pltpu.async_copy(k_hbm.at[sv], kb, sm)
        pltpu.async_copy(bias_hbm.at[pl.ds(base, CHUNK), :], bb, sm)

    def drain(ci, b):
        dv, sv, qb, kb, bb, sm = b
        base = wid * EPW + ci * CHUNK
        pltpu.make_async_copy(q_hbm.at[dv], qb, sm).wait()
        pltpu.make_async_copy(k_hbm.at[sv], kb, sm).wait()
        pltpu.make_async_copy(bias_hbm.at[pl.ds(base, CHUNK), :], bb, sm).wait()

    def compute(ci, b):
        dv, sv, qb, kb, bb, sm = b
        base = wid * EPW + ci * CHUNK

        def edge_body(i, ecarry):
            for u in range(2):
                e = i * 2 + u
                acc = qb[e, pl.ds(0, 16)] * kb[e, pl.ds(0, 16)]
                for j in range(1, W):
                    acc = acc + (qb[e, pl.ds(16 * j, 16)]
                                 * kb[e, pl.ds(16 * j, 16)])
                acc = acc + _take(acc, swap1)
                acc = acc + _take(acc, swap2)
                logits = _take(acc, heads) * SCALE + bb[e, :]
                exv[e, :] = jnp.where(lane < H, jnp.exp(logits), 0.0)
            return ecarry

        lax.fori_loop(0, CHUNK // 2, edge_body, 0)
        pltpu.sync_copy(exv, ex_hbm.at[pl.ds(base, CHUNK), :])
        pltpu.sync_copy(exv, nrm_sh.at[dv], add=True)

    fire(0, bufs[0])

    def pair_body(g, carry):
        ci = g * 2
        fire(ci + 1, bufs[1])
        drain(ci, bufs[0])
        compute(ci, bufs[0])

        @pl.when(g < NCHUNK // 2 - 1)
        def _():
            fire(ci + 2, bufs[0])

        drain(ci + 1, bufs[1])
        compute(ci + 1, bufs[1])
        return carry

    lax.fori_loop(0, NCHUNK // 2, pair_body, 0)
    plsc.subcore_barrier()

    pltpu.sync_copy(nrm_sh.at[pl.ds(s * RPS, RPS), :], stage)

    @pl.when(c == 0)
    def _():
        pltpu.sync_copy(stage, nrm0_hbm.at[pl.ds(s * RPS, RPS), :])

    @pl.when(c == 1)
    def _():
        pltpu.sync_copy(stage, nrm1_hbm.at[pl.ds(s * RPS, RPS), :])


_sc_pass_a = functools.partial(
    pl.kernel,
    _pass_a_body,
    out_type=(jax.ShapeDtypeStruct((EP, 16), jnp.float32),
              jax.ShapeDtypeStruct((NP, 16), jnp.float32),
              jax.ShapeDtypeStruct((NP, 16), jnp.float32)),
    mesh=_mesh,
    scratch_types=[
        pltpu.VMEM((CHUNK,), jnp.int32),
        pltpu.VMEM((CHUNK,), jnp.int32),
        pltpu.VMEM((CHUNK, D), jnp.float32),
        pltpu.VMEM((CHUNK, D), jnp.float32),
        pltpu.VMEM((CHUNK, 16), jnp.float32),
        pltpu.VMEM((CHUNK,), jnp.int32),
        pltpu.VMEM((CHUNK,), jnp.int32),
        pltpu.VMEM((CHUNK, D), jnp.float32),
        pltpu.VMEM((CHUNK, D), jnp.float32),
        pltpu.VMEM((CHUNK, 16), jnp.float32),
        pltpu.VMEM((CHUNK, 16), jnp.float32),
        pltpu.VMEM((RPS, 16), jnp.float32),
        pltpu.VMEM_SHARED((NP, 16), jnp.float32),
        pltpu.SemaphoreType.DMA,
        pltpu.SemaphoreType.DMA,
    ],
    compiler_params=_sc_params,
)()


def _pass_b_body(dst_hbm, src_hbm, vlo_hbm, vhi_hbm, ex_hbm, nrm0_hbm,
                 nrm1_hbm,
                 a0lo_hbm, a0hi_hbm, a1lo_hbm, a1hi_hbm,
                 dstv0, srcv0, vrows0, exr0, n0r0, n1r0,
                 dstv1, srcv1, vrows1, exr1, n0r1, n1r1,
                 wrows, stage, agg_sh, sem0, sem1):
    c = lax.axis_index("c")
    s = lax.axis_index("s")
    wid = s * 2 + c

    _, _, _, _, bcast = _lane_perms()
    bufs = ((dstv0, srcv0, vrows0, exr0, n0r0, n1r0, sem0),
            (dstv1, srcv1, vrows1, exr1, n0r1, n1r1, sem1))

    for v_hbm, out0_hbm, out1_hbm in ((vlo_hbm, a0lo_hbm, a1lo_hbm),
                                      (vhi_hbm, a0hi_hbm, a1hi_hbm)):
        def zero_row(r, carry):
            for j in range(HW):
                stage[r, j, :] = jnp.zeros((16,), jnp.float32)
            return carry
        lax.fori_loop(0, RPS, zero_row, 0)
        pltpu.sync_copy(stage, agg_sh.at[pl.ds(s * RPS, RPS), :, :])
        plsc.subcore_barrier()

        def fire(ci, b):
            dv, sv, vb, eb, n0b, n1b, sm = b
            base = wid * EPW + ci * CHUNK
            pltpu.sync_copy(dst_hbm.at[pl.ds(base, CHUNK)], dv)
            pltpu.sync_copy(src_hbm.at[pl.ds(base, CHUNK)], sv)
            pltpu.async_copy(v_hbm.at[sv], vb, sm)
            pltpu.async_copy(ex_hbm.at[pl.ds(base, CHUNK), :], eb, sm)
            pltpu.async_copy(nrm0_hbm.at[dv], n0b, sm)
            pltpu.async_copy(nrm1_hbm.at[dv], n1b, sm)

        def drain(ci, b):
            dv, sv, vb, eb, n0b, n1b, sm = b
            base = wid * EPW + ci * CHUNK
            pltpu.make_async_copy(v_hbm.at[sv], vb, sm).wait()
            pltpu.make_async_copy(ex_hbm.at[pl.ds(base, CHUNK), :], eb, sm).wait()
            pltpu.make_async_copy(nrm0_hbm.at[dv], n0b, sm).wait()
            pltpu.make_async_copy(nrm1_hbm.at[dv], n1b, sm).wait()

        def compute(ci, b):
            dv, sv, vb, eb, n0b, n1b, sm = b

            def edge_body(i, ecarry):
                for u in range(2):
                    e = i * 2 + u
                    nrm = n0b[e, :] + n1b[e, :]
                    attn = eb[e, :] / jnp.maximum(nrm, 1e-12)
                    att_b = _take(attn, bcast)
                    for j in range(HW):
                        wrows[e, j, :] = vb[e, j, :] * att_b
                return ecarry

            lax.fori_loop(0, CHUNK // 2, edge_body, 0)
            pltpu.sync_copy(wrows, agg_sh.at[dv], add=True)

        fire(0, bufs[0])

        def pair_body(g, carry):
            ci = g * 2
            fire(ci + 1, bufs[1])
            drain(ci, bufs[0])
            compute(ci, bufs[0])

            @pl.when(g < NCHUNK // 2 - 1)
            def _():
                fire(ci + 2, bufs[0])

            drain(ci + 1, bufs[1])
            compute(ci + 1, bufs[1])
            return carry

        lax.fori_loop(0, NCHUNK // 2, pair_body, 0)
        plsc.subcore_barrier()

        pltpu.sync_copy(agg_sh.at[pl.ds(s * RPS, RPS), :, :], stage)

        @pl.when(c == 0)
        def _():
            pltpu.sync_copy(stage, out0_hbm.at[pl.ds(s * RPS, RPS), :, :])

        @pl.when(c == 1)
        def _():
            pltpu.sync_copy(stage, out1_hbm.at[pl.ds(s * RPS, RPS), :, :])


_sc_pass_b = functools.partial(
    pl.kernel,
    _pass_b_body,
    out_type=tuple(jax.ShapeDtypeStruct((NP, HW, 16), jnp.float32)
                   for _ in range(4)),
    mesh=_mesh,
    scratch_types=[
        pltpu.VMEM((CHUNK,), jnp.int32),
        pltpu.VMEM((CHUNK,), jnp.int32),
        pltpu.VMEM((CHUNK, HW, 16), jnp.float32),
        pltpu.VMEM((CHUNK, 16), jnp.float32),
        pltpu.VMEM((CHUNK, 16), jnp.float32),
        pltpu.VMEM((CHUNK, 16), jnp.float32),
        pltpu.VMEM((CHUNK,), jnp.int32),
        pltpu.VMEM((CHUNK,), jnp.int32),
        pltpu.VMEM((CHUNK, HW, 16), jnp.float32),
        pltpu.VMEM((CHUNK, 16), jnp.float32),
        pltpu.VMEM((CHUNK, 16), jnp.float32),
        pltpu.VMEM((CHUNK, 16), jnp.float32),
        pltpu.VMEM((CHUNK, HW, 16), jnp.float32),
        pltpu.VMEM((RPS, HW, 16), jnp.float32),
        pltpu.VMEM_SHARED((NP, HW, 16), jnp.float32),
        pltpu.SemaphoreType.DMA,
        pltpu.SemaphoreType.DMA,
    ],
    compiler_params=_sc_params,
)()


def kernel(node_features, edge_index, edge_features, Wq, bq, Wk, bk, Wv, bv,
           Wb1, bb1, Wb2, bb2, Wo, bo, Wf1, bf1, Wf2, bf2, g1, be1, g2, be2):
    cols = np.array(_COLS, np.int32)
    x_pad = jnp.zeros((NP, D), jnp.float32).at[:N].set(node_features)
    pad_e = EP - E
    src_pad = jnp.concatenate([edge_index[0], jnp.zeros((pad_e,), jnp.int32)])
    dst_pad = jnp.concatenate([edge_index[1], jnp.full((pad_e,), N, jnp.int32)])
    ef_pad = jnp.zeros((EP, DE), jnp.float32).at[:E].set(edge_features)

    q, k, v = _tc_qkv(x_pad, Wq[:, cols], bq[cols], Wk[:, cols], bk[cols],
                      Wv[:, cols], bv[cols])
    v = v.reshape(NP, W, 16)
    bias = _tc_bias(ef_pad, Wb1, bb1, Wb2, bb2)

    ex, nrm0, nrm1 = _sc_pass_a(dst_pad, src_pad, q, k, bias)
    a0lo, a0hi, a1lo, a1hi = _sc_pass_b(
        dst_pad, src_pad, v[:, :HW, :], v[:, HW:, :], ex, nrm0, nrm1)
    agg0 = jnp.concatenate([a0lo.reshape(NP, D // 2), a0hi.reshape(NP, D // 2)], axis=1)
    agg1 = jnp.concatenate([a1lo.reshape(NP, D // 2), a1hi.reshape(NP, D // 2)], axis=1)

    out = _tc_final(agg0, agg1, x_pad,
                    Wo[cols, :], bo, Wf1, bf1, Wf2, bf2, g1, be1, g2, be2)
    return out[:N]


# speedup vs baseline: 15.3017x; 1.0293x over previous
import functools

import jax
import jax.numpy as jnp
import numpy as np
from jax import lax
from jax.experimental import pallas as pl
from jax.experimental.pallas import tpu as pltpu
from jax.experimental.pallas import tpu_sc as plsc

N = 10000
E = 320000
D = 128
H = 4
DH = D // H
DE = 16
SCALE = 1.0 / (DH ** 0.5)

NP = 10240
NW = 32
CHUNK = 128
NCHUNK = 80
EPW = CHUNK * NCHUNK
EP = EPW * NW
RPS = NP // 16
W = D // 16
HW = W // 2

_COLS = [32 * h + 4 * j + t for j in range(W) for h in range(H) for t in range(4)]

_mesh = plsc.VectorSubcoreMesh(core_axis_name="c", subcore_axis_name="s")
_sc_params = pltpu.CompilerParams(use_tc_tiling_on_sc=False)


def _silu(x):
    return x * (1.0 / (1.0 + jnp.exp(-x)))


def _ln(x, g, b):
    mu = jnp.mean(x, axis=-1, keepdims=True)
    var = jnp.mean((x - mu) ** 2, axis=-1, keepdims=True)
    return (x - mu) / jnp.sqrt(var + 1e-5) * g + b


def _qkv_body(x_ref, wq_ref, bq_ref, wk_ref, bk_ref, wv_ref, bv_ref,
              q_ref, k_ref, v_ref):
    xb = x_ref[...]
    q_ref[...] = jnp.dot(xb, wq_ref[...], preferred_element_type=jnp.float32) + bq_ref[...][None, :]
    k_ref[...] = jnp.dot(xb, wk_ref[...], preferred_element_type=jnp.float32) + bk_ref[...][None, :]
    v_ref[...] = jnp.dot(xb, wv_ref[...], preferred_element_type=jnp.float32) + bv_ref[...][None, :]


def _tc_qkv(x_pad, Wq, bq, Wk, bk, Wv, bv):
    blk = 512
    grid = NP // blk
    whole2 = pl.BlockSpec((D, D), lambda i: (0, 0))
    whole1 = pl.BlockSpec((D,), lambda i: (0,))
    rows = pl.BlockSpec((blk, D), lambda i: (i, 0))
    return pl.pallas_call(
        _qkv_body,
        grid=(grid,),
        in_specs=[rows, whole2, whole1, whole2, whole1, whole2, whole1],
        out_specs=[rows, rows, rows],
        out_shape=[jax.ShapeDtypeStruct((NP, D), jnp.float32)] * 3,
    )(x_pad, Wq, bq, Wk, bk, Wv, bv)


def _bias_body(ef_ref, w1_ref, b1_ref, w2_ref, b2_ref, o_ref):
    h1 = jnp.dot(ef_ref[...], w1_ref[...], preferred_element_type=jnp.float32) + b1_ref[...][None, :]
    h1 = _silu(h1)
    b = jnp.dot(h1, w2_ref[...], preferred_element_type=jnp.float32) + b2_ref[...][None, :]
    o_ref[...] = jnp.concatenate(
        [b, jnp.zeros((b.shape[0], 16 - H), jnp.float32)], axis=1)


def _tc_bias(ef_pad, Wb1, bb1, Wb2, bb2):
    blk = 4096
    grid = EP // blk
    return pl.pallas_call(
        _bias_body,
        grid=(grid,),
        in_specs=[
            pl.BlockSpec((blk, DE), lambda i: (i, 0)),
            pl.BlockSpec((DE, H), lambda i: (0, 0)),
            pl.BlockSpec((H,), lambda i: (0,)),
            pl.BlockSpec((H, H), lambda i: (0, 0)),
            pl.BlockSpec((H,), lambda i: (0,)),
        ],
        out_specs=pl.BlockSpec((blk, 16), lambda i: (i, 0)),
        out_shape=jax.ShapeDtypeStruct((EP, 16), jnp.float32),
    )(ef_pad, Wb1, bb1, Wb2, bb2)


def _final_body(a0_ref, a1_ref, x_ref, wo_ref, bo_ref, wf1_ref, bf1_ref,
                wf2_ref, bf2_ref, g1_ref, be1_ref, g2_ref, be2_ref, o_ref):
    agg = a0_ref[...] + a1_ref[...]
    o = jnp.dot(agg, wo_ref[...], preferred_element_type=jnp.float32) + bo_ref[...][None, :]
    x1 = _ln(x_ref[...] + o, g1_ref[...][None, :], be1_ref[...][None, :])
    h = _silu(jnp.dot(x1, wf1_ref[...], preferred_element_type=jnp.float32) + bf1_ref[...][None, :])
    f = jnp.dot(h, wf2_ref[...], preferred_element_type=jnp.float32) + bf2_ref[...][None, :]
    o_ref[...] = _ln(x1 + f, g2_ref[...][None, :], be2_ref[...][None, :])


def _tc_final(a0, a1, x_pad, Wo, bo, Wf1, bf1, Wf2, bf2, g1, be1, g2, be2):
    blk = 512
    grid = NP // blk
    rows = pl.BlockSpec((blk, D), lambda i: (i, 0))
    w128 = pl.BlockSpec((D,), lambda i: (0,))
    return pl.pallas_call(
        _final_body,
        grid=(grid,),
        in_specs=[
            rows, rows, rows,
            pl.BlockSpec((D, D), lambda i: (0, 0)), w128,
            pl.BlockSpec((D, 2 * D), lambda i: (0, 0)),
            pl.BlockSpec((2 * D,), lambda i: (0,)),
            pl.BlockSpec((2 * D, D), lambda i: (0, 0)), w128,
            w128, w128, w128, w128,
        ],
        out_specs=rows,
        out_shape=jax.ShapeDtypeStruct((NP, D), jnp.float32),
    )(a0, a1, x_pad, Wo, bo, Wf1, bf1, Wf2, bf2, g1, be1, g2, be2)


def _take(x, idx):
    return jnp.take_along_axis(x, idx, axis=0, mode="promise_in_bounds")


def _lane_perms():
    lane = lax.iota(jnp.int32, 16)
    swap1 = lane ^ 1
    swap2 = lane ^ 2
    heads = (lane & 3) * 4
    bcast = lane >> 2
    return lane, swap1, swap2, heads, bcast


def _pass_a_body(dst_hbm, src_hbm, q_hbm, k_hbm, bias_hbm,
                 ex_hbm, nrm0_hbm, nrm1_hbm,
                 dstv0, srcv0, qrows0, krows0, brows0,
                 dstv1, srcv1, qrows1, krows1, brows1,
                 exv, stage, nrm_sh, sem0, sem1):
    c = lax.axis_index("c")
    s = lax.axis_index("s")
    wid = s * 2 + c

    def zero_row(r, carry):
        stage[r, :] = jnp.zeros((16,), jnp.float32)
        return carry
    lax.fori_loop(0, RPS, zero_row, 0)
    pltpu.sync_copy(stage, nrm_sh.at[pl.ds(s * RPS, RPS), :])
    plsc.subcore_barrier()

    lane, swap1, swap2, heads, _ = _lane_perms()
    bufs = ((dstv0, srcv0, qrows0, krows0, brows0, sem0),
            (dstv1, srcv1, qrows1, krows1, brows1, sem1))

    def fire(ci, b):
        dv, sv, qb, kb, bb, sm = b
        base = wid * EPW + ci * CHUNK
        pltpu.sync_copy(dst_hbm.at[pl.ds(base, CHUNK)], dv)
        pltpu.sync_copy(src_hbm.at[pl.ds(base, CHUNK)], sv)
        pltpu.async_copy(q_hbm.at[dv], qb, sm)
        pltpu.async_copy(k_hbm.at[sv], kb, sm)
        pltpu.async_copy(bias_hbm.at[pl.ds(base, CHUNK), :], bb, sm)

    def drain(ci, b):
        dv, sv, qb, kb, bb, sm = b
        base = wid * EPW + ci * CHUNK
        pltpu.make_async_copy(q_hbm.at[dv], qb, sm).wait()
        pltpu.make_async_copy(k_hbm.at[sv], kb, sm).wait()
        pltpu.make_async_copy(bias_hbm.at[pl.ds(base, CHUNK), :], bb, sm).wait()

    def compute(ci, b):
        dv, sv, qb, kb, bb, sm = b
        base = wid * EPW + ci * CHUNK

        def edge_body(i, ecarry):
            for u in range(2):
                e = i * 2 + u
                acc = qb[e, pl.ds(0, 16)] * kb[e, pl.ds(0, 16)]
                for j in range(1, W):
                    acc = acc + (qb[e, pl.ds(16 * j, 16)]
                                 * kb[e, pl.ds(16 * j, 16)])
                acc = acc + _take(acc, swap1)
                acc = acc + _take(acc, swap2)
                logits = _take(acc, heads) * SCALE + bb[e, :]
                exv[e, :] = jnp.where(lane < H, jnp.exp(logits), 0.0)
            return ecarry

        lax.fori_loop(0, CHUNK // 2, edge_body, 0)
        pltpu.sync_copy(exv, ex_hbm.at[pl.ds(base, CHUNK), :])
        pltpu.sync_copy(exv, nrm_sh.at[dv], add=True)

    fire(0, bufs[0])

    def pair_body(g, carry):
        ci = g * 2
        fire(ci + 1, bufs[1])
        drain(ci, bufs[0])
        compute(ci, bufs[0])

        @pl.when(g < NCHUNK // 2 - 1)
        def _():
            fire(ci + 2, bufs[0])

        drain(ci + 1, bufs[1])
        compute(ci + 1, bufs[1])
        return carry

    lax.fori_loop(0, NCHUNK // 2, pair_body, 0)
    plsc.subcore_barrier()

    pltpu.sync_copy(nrm_sh.at[pl.ds(s * RPS, RPS), :], stage)

    @pl.when(c == 0)
    def _():
        pltpu.sync_copy(stage, nrm0_hbm.at[pl.ds(s * RPS, RPS), :])

    @pl.when(c == 1)
    def _():
        pltpu.sync_copy(stage, nrm1_hbm.at[pl.ds(s * RPS, RPS), :])


_sc_pass_a = functools.partial(
    pl.kernel,
    _pass_a_body,
    out_type=(jax.ShapeDtypeStruct((EP, 16), jnp.float32),
              jax.ShapeDtypeStruct((NP, 16), jnp.float32),
              jax.ShapeDtypeStruct((NP, 16), jnp.float32)),
    mesh=_mesh,
    scratch_types=[
        pltpu.VMEM((CHUNK,), jnp.int32),
        pltpu.VMEM((CHUNK,), jnp.int32),
        pltpu.VMEM((CHUNK, D), jnp.float32),
        pltpu.VMEM((CHUNK, D), jnp.float32),
        pltpu.VMEM((CHUNK, 16), jnp.float32),
        pltpu.VMEM((CHUNK,), jnp.int32),
        pltpu.VMEM((CHUNK,), jnp.int32),
        pltpu.VMEM((CHUNK, D), jnp.float32),
        pltpu.VMEM((CHUNK, D), jnp.float32),
        pltpu.VMEM((CHUNK, 16), jnp.float32),
        pltpu.VMEM((CHUNK, 16), jnp.float32),
        pltpu.VMEM((RPS, 16), jnp.float32),
        pltpu.VMEM_SHARED((NP, 16), jnp.float32),
        pltpu.SemaphoreType.DMA,
        pltpu.SemaphoreType.DMA,
    ],
    compiler_params=_sc_params,
)()


def _pass_b_body(dst_hbm, src_hbm, vlo_hbm, vhi_hbm, ex_hbm, nrm0_hbm,
                 nrm1_hbm,
                 a0lo_hbm, a0hi_hbm, a1lo_hbm, a1hi_hbm,
                 dstv0, srcv0, vrows0, exr0, n0r0, n1r0,
                 dstv1, srcv1, vrows1, exr1, n0r1, n1r1,
                 wrows0, wrows1, sidx0, sidx1,
                 stage, agg_sh, sem0, sem1, ssem0, ssem1):
    c = lax.axis_index("c")
    s = lax.axis_index("s")
    wid = s * 2 + c

    _, _, _, _, bcast = _lane_perms()
    bufs = ((dstv0, srcv0, vrows0, exr0, n0r0, n1r0, sem0),
            (dstv1, srcv1, vrows1, exr1, n0r1, n1r1, sem1))
    sbufs = ((wrows0, sidx0, ssem0), (wrows1, sidx1, ssem1))

    for v_hbm, out0_hbm, out1_hbm in ((vlo_hbm, a0lo_hbm, a1lo_hbm),
                                      (vhi_hbm, a0hi_hbm, a1hi_hbm)):
        def zero_row(r, carry):
            for j in range(HW):
                stage[r, j, :] = jnp.zeros((16,), jnp.float32)
            return carry
        lax.fori_loop(0, RPS, zero_row, 0)
        pltpu.sync_copy(stage, agg_sh.at[pl.ds(s * RPS, RPS), :, :])
        plsc.subcore_barrier()

        def fire(ci, b):
            dv, sv, vb, eb, n0b, n1b, sm = b
            base = wid * EPW + ci * CHUNK
            pltpu.sync_copy(dst_hbm.at[pl.ds(base, CHUNK)], dv)
            pltpu.sync_copy(src_hbm.at[pl.ds(base, CHUNK)], sv)
            pltpu.async_copy(v_hbm.at[sv], vb, sm)
            pltpu.async_copy(ex_hbm.at[pl.ds(base, CHUNK), :], eb, sm)
            pltpu.async_copy(nrm0_hbm.at[dv], n0b, sm)
            pltpu.async_copy(nrm1_hbm.at[dv], n1b, sm)

        def drain(ci, b):
            dv, sv, vb, eb, n0b, n1b, sm = b
            base = wid * EPW + ci * CHUNK
            pltpu.make_async_copy(v_hbm.at[sv], vb, sm).wait()
            pltpu.make_async_copy(ex_hbm.at[pl.ds(base, CHUNK), :], eb, sm).wait()
            pltpu.make_async_copy(nrm0_hbm.at[dv], n0b, sm).wait()
            pltpu.make_async_copy(nrm1_hbm.at[dv], n1b, sm).wait()

        def compute(g, b, sb):
            dv, sv, vb, eb, n0b, n1b, sm = b
            wb, sx, ssm = sb

            @pl.when(g > 0)
            def _():
                pltpu.make_async_copy(wb, agg_sh.at[sx], ssm).wait()

            def edge_body(i, ecarry):
                for u in range(2):
                    e = i * 2 + u
                    nrm = n0b[e, :] + n1b[e, :]
                    attn = eb[e, :] / jnp.maximum(nrm, 1e-12)
                    att_b = _take(attn, bcast)
                    for j in range(HW):
                        wb[e, j, :] = vb[e, j, :] * att_b
                return ecarry

            lax.fori_loop(0, CHUNK // 2, edge_body, 0)
            for i16 in range(CHUNK // 16):
                sx[pl.ds(i16 * 16, 16)] = dv[pl.ds(i16 * 16, 16)]
            pltpu.async_copy(wb, agg_sh.at[sx], ssm, add=True)

        fire(0, bufs[0])

        def pair_body(g, carry):
            ci = g * 2
            fire(ci + 1, bufs[1])
            drain(ci, bufs[0])
            compute(g, bufs[0], sbufs[0])

            @pl.when(g < NCHUNK // 2 - 1)
            def _():
                fire(ci + 2, bufs[0])

            drain(ci + 1, bufs[1])
            compute(g, bufs[1], sbufs[1])
            return carry

        lax.fori_loop(0, NCHUNK // 2, pair_body, 0)
        pltpu.make_async_copy(wrows0, agg_sh.at[sidx0], ssem0).wait()
        pltpu.make_async_copy(wrows1, agg_sh.at[sidx1], ssem1).wait()
        plsc.subcore_barrier()

        pltpu.sync_copy(agg_sh.at[pl.ds(s * RPS, RPS), :, :], stage)

        @pl.when(c == 0)
        def _():
            pltpu.sync_copy(stage, out0_hbm.at[pl.ds(s * RPS, RPS), :, :])

        @pl.when(c == 1)
        def _():
            pltpu.sync_copy(stage, out1_hbm.at[pl.ds(s * RPS, RPS), :, :])


_sc_pass_b = functools.partial(
    pl.kernel,
    _pass_b_body,
    out_type=tuple(jax.ShapeDtypeStruct((NP, HW, 16), jnp.float32)
                   for _ in range(4)),
    mesh=_mesh,
    scratch_types=[
        pltpu.VMEM((CHUNK,), jnp.int32),
        pltpu.VMEM((CHUNK,), jnp.int32),
        pltpu.VMEM((CHUNK, HW, 16), jnp.float32),
        pltpu.VMEM((CHUNK, 16), jnp.float32),
        pltpu.VMEM((CHUNK, 16), jnp.float32),
        pltpu.VMEM((CHUNK, 16), jnp.float32),
        pltpu.VMEM((CHUNK,), jnp.int32),
        pltpu.VMEM((CHUNK,), jnp.int32),
        pltpu.VMEM((CHUNK, HW, 16), jnp.float32),
        pltpu.VMEM((CHUNK, 16), jnp.float32),
        pltpu.VMEM((CHUNK, 16), jnp.float32),
        pltpu.VMEM((CHUNK, 16), jnp.float32),
        pltpu.VMEM((CHUNK, HW, 16), jnp.float32),
        pltpu.VMEM((CHUNK, HW, 16), jnp.float32),
        pltpu.VMEM((CHUNK,), jnp.int32),
        pltpu.VMEM((CHUNK,), jnp.int32),
        pltpu.VMEM((RPS, HW, 16), jnp.float32),
        pltpu.VMEM_SHARED((NP, HW, 16), jnp.float32),
        pltpu.SemaphoreType.DMA,
        pltpu.SemaphoreType.DMA,
        pltpu.SemaphoreType.DMA,
        pltpu.SemaphoreType.DMA,
    ],
    compiler_params=_sc_params,
)()


def kernel(node_features, edge_index, edge_features, Wq, bq, Wk, bk, Wv, bv,
           Wb1, bb1, Wb2, bb2, Wo, bo, Wf1, bf1, Wf2, bf2, g1, be1, g2, be2):
    cols = np.array(_COLS, np.int32)
    x_pad = jnp.zeros((NP, D), jnp.float32).at[:N].set(node_features)
    pad_e = EP - E
    src_pad = jnp.concatenate([edge_index[0], jnp.zeros((pad_e,), jnp.int32)])
    dst_pad = jnp.concatenate([edge_index[1], jnp.full((pad_e,), N, jnp.int32)])
    ef_pad = jnp.zeros((EP, DE), jnp.float32).at[:E].set(edge_features)

    q, k, v = _tc_qkv(x_pad, Wq[:, cols], bq[cols], Wk[:, cols], bk[cols],
                      Wv[:, cols], bv[cols])
    v = v.reshape(NP, W, 16)
    bias = _tc_bias(ef_pad, Wb1, bb1, Wb2, bb2)

    ex, nrm0, nrm1 = _sc_pass_a(dst_pad, src_pad, q, k, bias)
    a0lo, a0hi, a1lo, a1hi = _sc_pass_b(
        dst_pad, src_pad, v[:, :HW, :], v[:, HW:, :], ex, nrm0, nrm1)
    agg0 = jnp.concatenate([a0lo.reshape(NP, D // 2), a0hi.reshape(NP, D // 2)], axis=1)
    agg1 = jnp.concatenate([a1lo.reshape(NP, D // 2), a1hi.reshape(NP, D // 2)], axis=1)

    out = _tc_final(agg0, agg1, x_pad,
                    Wo[cols, :], bo, Wf1, bf1, Wf2, bf2, g1, be1, g2, be2)
    return out[:N]
